# baseline (speedup 1.0000x reference)
import jax
import jax.numpy as jnp
from jax.experimental import pallas as pl
import jax.experimental.pallas.tpu as pltpu

RES = 4
L_PTS = 10
L_DIR = 4
HID = 32
BLK = 2048


def _pe_parts(x, L):
    parts = [x]
    for l in range(L):
        xb = x * (2.0 ** l)
        parts.append(jnp.sin(xb))
        parts.append(jnp.cos(xb))
    return jnp.concatenate(parts, axis=-1)


def _network_kernel(pts_ref, dirs_ref,
                    W1_ref, b1_ref, W2_ref, b2_ref, Wf_ref, bf_ref,
                    Ws_ref, bs_ref, Wv_ref, bv_ref, Wr_ref, br_ref,
                    rgb_ref, sigma_ref,
                    embp_scr, embd_scr, flat_scr):
    e = pl.program_id(1)

    @pl.when(e == 0)
    def _init():
        pts = pts_ref[...]
        embp_scr[...] = _pe_parts(pts, L_PTS)
        embd_scr[...] = _pe_parts(dirs_ref[...], L_DIR)
        scaled = (pts + 1.0) * (0.5 * RES)
        clamped = jnp.clip(scaled, 0.0, RES - 1.0)
        idx3 = clamped.astype(jnp.int32)
        flat = idx3[:, 0] * (RES * RES) + idx3[:, 1] * RES + idx3[:, 2]
        flat_scr[...] = flat[:, None]
        rgb_ref[...] = jnp.zeros_like(rgb_ref)
        sigma_ref[...] = jnp.zeros_like(sigma_ref)

    emb_p = embp_scr[...]
    emb_d = embd_scr[...]

    h = jax.nn.relu(jnp.dot(emb_p, W1_ref[0], preferred_element_type=jnp.float32)
                    + b1_ref[0])
    h = jax.nn.relu(jnp.dot(h, W2_ref[0], preferred_element_type=jnp.float32)
                    + b2_ref[0])
    sig = jnp.dot(h, Ws_ref[0], preferred_element_type=jnp.float32) + bs_ref[0]
    feat = jnp.dot(h, Wf_ref[0], preferred_element_type=jnp.float32) + bf_ref[0]
    h2 = jnp.concatenate([feat, emb_d], axis=-1)
    h2 = jax.nn.relu(jnp.dot(h2, Wv_ref[0], preferred_element_type=jnp.float32)
                     + bv_ref[0])
    rgb = jnp.dot(h2, Wr_ref[0], preferred_element_type=jnp.float32) + br_ref[0]

    mask = (flat_scr[...] == e).astype(jnp.float32)
    rgb_ref[...] += mask * rgb
    sigma_ref[...] += mask * sig


def kernel(pts, viewdirs, W1, b1, W2, b2, Wf, bf, Ws, bs, Wv, bv, Wr, br):
    N_rays, N_samp, _ = pts.shape
    N = N_rays * N_samp
    E = W1.shape[0]
    PB = N // BLK
    pts_flat = pts.reshape(N, 3)
    dirs_flat = jnp.broadcast_to(viewdirs[:, None, :], (N_rays, N_samp, 3)).reshape(N, 3)

    D_P = 3 + 6 * L_PTS
    D_D = 3 + 6 * L_DIR

    b1r = b1[:, None, :]
    b2r = b2[:, None, :]
    bfr = bf[:, None, :]
    bsr = bs[:, None, :]
    bvr = bv[:, None, :]
    brr = br[:, None, :]

    pt_blk = lambda d: pl.BlockSpec((BLK, d), lambda p, e: (p, 0))
    per_e2 = lambda d0, d1: pl.BlockSpec((1, d0, d1), lambda p, e: (e, 0, 0))
    per_e1 = lambda d0: pl.BlockSpec((1, 1, d0), lambda p, e: (e, 0, 0))

    rgb, sigma = pl.pallas_call(
        _network_kernel,
        grid=(PB, E),
        in_specs=[
            pt_blk(3), pt_blk(3),
            per_e2(D_P, HID), per_e1(HID),
            per_e2(HID, HID), per_e1(HID),
            per_e2(HID, HID), per_e1(HID),
            per_e2(HID, 1), per_e1(1),
            per_e2(D_D + HID, HID), per_e1(HID),
            per_e2(HID, 3), per_e1(3),
        ],
        out_specs=[pt_blk(3), pt_blk(1)],
        out_shape=[
            jax.ShapeDtypeStruct((N, 3), jnp.float32),
            jax.ShapeDtypeStruct((N, 1), jnp.float32),
        ],
        scratch_shapes=[
            pltpu.VMEM((BLK, D_P), jnp.float32),
            pltpu.VMEM((BLK, D_D), jnp.float32),
            pltpu.VMEM((BLK, 1), jnp.int32),
        ],
    )(pts_flat, dirs_flat, W1, b1r, W2, b2r, Wf, bfr, Ws, bsr, Wv, bvr, Wr, brr)

    return rgb.reshape(N_rays, N_samp, 3), sigma.reshape(N_rays, N_samp, 1)


# speedup vs baseline: 3.6068x; 3.6068x over previous
import jax
import jax.numpy as jnp
from jax.experimental import pallas as pl
import jax.experimental.pallas.tpu as pltpu

RES = 4
L_PTS = 10
L_DIR = 4
HID = 32
BLK = 2048


def _pe_parts_t(x, L):
    parts = [x]
    for l in range(L):
        xb = x * (2.0 ** l)
        parts.append(jnp.sin(xb))
        parts.append(jnp.cos(xb))
    return jnp.concatenate(parts, axis=0)


def _network_kernel(pts_ref, dirs_ref,
                    W1_ref, b1_ref, W2_ref, b2_ref, Wf_ref, bf_ref,
                    Ws_ref, bs_ref, Wv_ref, bv_ref, Wr_ref, br_ref,
                    rgb_ref, sigma_ref,
                    embp_scr, embd_scr, flat_scr):
    e = pl.program_id(1)

    @pl.when(e == 0)
    def _init():
        pts = pts_ref[...]
        embp_scr[...] = _pe_parts_t(pts, L_PTS)
        embd_scr[...] = _pe_parts_t(dirs_ref[...], L_DIR)
        scaled = (pts + 1.0) * (0.5 * RES)
        clamped = jnp.clip(scaled, 0.0, RES - 1.0)
        idx3 = clamped.astype(jnp.int32)
        flat = (idx3[0:1, :] * (RES * RES) + idx3[1:2, :] * RES
                + idx3[2:3, :])
        flat_scr[...] = flat
        rgb_ref[...] = jnp.zeros_like(rgb_ref)
        sigma_ref[...] = jnp.zeros_like(sigma_ref)

    emb_p = embp_scr[...]
    emb_d = embd_scr[...]

    dot = lambda a, b: jnp.dot(a, b, preferred_element_type=jnp.float32)
    h = jax.nn.relu(dot(W1_ref[0], emb_p) + b1_ref[0])
    h = jax.nn.relu(dot(W2_ref[0], h) + b2_ref[0])
    sig = dot(Ws_ref[0], h) + bs_ref[0]
    feat = dot(Wf_ref[0], h) + bf_ref[0]
    h2 = jnp.concatenate([feat, emb_d], axis=0)
    h2 = jax.nn.relu(dot(Wv_ref[0], h2) + bv_ref[0])
    rgb = dot(Wr_ref[0], h2) + br_ref[0]

    mask = (flat_scr[...] == e).astype(jnp.float32)
    rgb_ref[...] += mask * rgb
    sigma_ref[...] += mask * sig


def kernel(pts, viewdirs, W1, b1, W2, b2, Wf, bf, Ws, bs, Wv, bv, Wr, br):
    N_rays, N_samp, _ = pts.shape
    N = N_rays * N_samp
    E = W1.shape[0]
    PB = N // BLK
    pts_t = pts.reshape(N, 3).T
    dirs_t = jnp.broadcast_to(viewdirs[:, None, :], (N_rays, N_samp, 3)).reshape(N, 3).T

    D_P = 3 + 6 * L_PTS
    D_D = 3 + 6 * L_DIR

    W1t = jnp.swapaxes(W1, 1, 2)
    W2t = jnp.swapaxes(W2, 1, 2)
    Wft = jnp.swapaxes(Wf, 1, 2)
    Wst = jnp.swapaxes(Ws, 1, 2)
    Wvt = jnp.swapaxes(Wv, 1, 2)
    Wrt = jnp.swapaxes(Wr, 1, 2)
    b1c = b1[:, :, None]
    b2c = b2[:, :, None]
    bfc = bf[:, :, None]
    bsc = bs[:, :, None]
    bvc = bv[:, :, None]
    brc = br[:, :, None]

    pt_blk = lambda d: pl.BlockSpec((d, BLK), lambda p, e: (0, p))
    per_e = lambda d0, d1: pl.BlockSpec((1, d0, d1), lambda p, e: (e, 0, 0))

    rgb_t, sigma_t = pl.pallas_call(
        _network_kernel,
        grid=(PB, E),
        in_specs=[
            pt_blk(3), pt_blk(3),
            per_e(HID, D_P), per_e(HID, 1),
            per_e(HID, HID), per_e(HID, 1),
            per_e(HID, HID), per_e(HID, 1),
            per_e(1, HID), per_e(1, 1),
            per_e(HID, D_D + HID), per_e(HID, 1),
            per_e(3, HID), per_e(3, 1),
        ],
        out_specs=[pt_blk(3), pt_blk(1)],
        out_shape=[
            jax.ShapeDtypeStruct((3, N), jnp.float32),
            jax.ShapeDtypeStruct((1, N), jnp.float32),
        ],
        scratch_shapes=[
            pltpu.VMEM((D_P, BLK), jnp.float32),
            pltpu.VMEM((D_D, BLK), jnp.float32),
            pltpu.VMEM((1, BLK), jnp.int32),
        ],
    )(pts_t, dirs_t, W1t, b1c, W2t, b2c, Wft, bfc, Wst, bsc, Wvt, bvc, Wrt, brc)

    rgb = rgb_t.T.reshape(N_rays, N_samp, 3)
    sigma = sigma_t.T.reshape(N_rays, N_samp, 1)
    return rgb, sigma


# speedup vs baseline: 5.2772x; 1.4631x over previous
import functools

import jax
import jax.numpy as jnp
from jax import lax
from jax.experimental import pallas as pl
import jax.experimental.pallas.tpu as pltpu

RES = 4
L_PTS = 10
L_DIR = 4
HID = 32
E = 64
TSZ = 256


def _pe_parts_t(x, L):
    parts = [x]
    for l in range(L):
        xb = x * (2.0 ** l)
        parts.append(jnp.sin(xb))
        parts.append(jnp.cos(xb))
    return jnp.concatenate(parts, axis=0)


def _mlp_kernel(te_ref, used_ref, rows_ref,
                W1_ref, b1_ref, W2_ref, b2_ref, Wf_ref, bf_ref,
                Ws_ref, bs_ref, Wv_ref, bv_ref, Wr_ref, br_ref,
                out_ref):
    t = pl.program_id(0)

    @pl.when(t < used_ref[0])
    def _compute():
        rt = rows_ref[...]
        x = rt[0:3, :]
        d = rt[3:6, :]
        emb_p = _pe_parts_t(x, L_PTS)
        emb_d = _pe_parts_t(d, L_DIR)

        dot = lambda a, b: jnp.dot(a, b, preferred_element_type=jnp.float32)
        h = jax.nn.relu(dot(W1_ref[0], emb_p) + b1_ref[0])
        h = jax.nn.relu(dot(W2_ref[0], h) + b2_ref[0])
        sig = dot(Ws_ref[0], h) + bs_ref[0]
        feat = dot(Wf_ref[0], h) + bf_ref[0]
        h2 = jnp.concatenate([feat, emb_d], axis=0)
        h2 = jax.nn.relu(dot(Wv_ref[0], h2) + bv_ref[0])
        rgb = dot(Wr_ref[0], h2) + br_ref[0]

        out_ref[...] = jnp.concatenate(
            [rgb, sig, jnp.zeros((4, rt.shape[1]), jnp.float32)], axis=0)


def _grouped_mlp(sorted_t, te, used,
                 W1t, b1c, W2t, b2c, Wft, bfc, Wst, bsc, Wvt, bvc, Wrt, brc):
    n_pad = sorted_t.shape[1]
    tiles = n_pad // TSZ

    D_P = 3 + 6 * L_PTS
    D_D = 3 + 6 * L_DIR

    blk = pl.BlockSpec((8, TSZ), lambda t, te_r, used_r: (0, t))
    per_e = lambda d0, d1: pl.BlockSpec(
        (1, d0, d1), lambda t, te_r, used_r: (te_r[t], 0, 0))

    grid_spec = pltpu.PrefetchScalarGridSpec(
        num_scalar_prefetch=2,
        grid=(tiles,),
        in_specs=[
            blk,
            per_e(HID, D_P), per_e(HID, 1),
            per_e(HID, HID), per_e(HID, 1),
            per_e(HID, HID), per_e(HID, 1),
            per_e(1, HID), per_e(1, 1),
            per_e(HID, D_D + HID), per_e(HID, 1),
            per_e(3, HID), per_e(3, 1),
        ],
        out_specs=blk,
    )
    return pl.pallas_call(
        _mlp_kernel,
        grid_spec=grid_spec,
        out_shape=jax.ShapeDtypeStruct((8, n_pad), jnp.float32),
    )(te, used, sorted_t,
      W1t, b1c, W2t, b2c, Wft, bfc, Wst, bsc, Wvt, bvc, Wrt, brc)


def kernel(pts, viewdirs, W1, b1, W2, b2, Wf, bf, Ws, bs, Wv, bv, Wr, br):
    N_rays, N_samp, _ = pts.shape
    N = N_rays * N_samp
    pts_flat = pts.reshape(N, 3)
    dirs_flat = jnp.broadcast_to(viewdirs[:, None, :], (N_rays, N_samp, 3)).reshape(N, 3)

    tiles = N // TSZ + E
    n_pad = tiles * TSZ

    comb = jnp.concatenate(
        [pts_flat, dirs_flat, jnp.zeros((N, 2), jnp.float32)], axis=1)

    scaled = (pts_flat + 1.0) * (0.5 * RES)
    idx3 = jnp.clip(scaled, 0.0, RES - 1.0).astype(jnp.int32)
    flat = idx3[:, 0] * (RES * RES) + idx3[:, 1] * RES + idx3[:, 2]

    oh = (flat[None, :] == jnp.arange(E, dtype=jnp.int32)[:, None])
    counts = oh.sum(axis=1).astype(jnp.int32)
    seg_tiles = (counts + TSZ - 1) // TSZ
    tile_start = jnp.cumsum(seg_tiles) - seg_tiles
    padded_start = tile_start * TSZ
    occ = jnp.cumsum(oh.astype(jnp.int32), axis=1) - 1
    rank = occ[flat, jnp.arange(N)]
    pos = padded_start[flat] + rank
    sorted_tab = jnp.zeros((n_pad, 8), jnp.float32).at[pos].set(comb)
    te = (tile_start[None, :] <= jnp.arange(tiles, dtype=jnp.int32)[:, None]
          ).sum(axis=1).astype(jnp.int32) - 1
    used = (tile_start[E - 1] + seg_tiles[E - 1]).astype(jnp.int32)[None]

    W1t = jnp.swapaxes(W1, 1, 2)
    W2t = jnp.swapaxes(W2, 1, 2)
    Wft = jnp.swapaxes(Wf, 1, 2)
    Wst = jnp.swapaxes(Ws, 1, 2)
    Wvt = jnp.swapaxes(Wv, 1, 2)
    Wrt = jnp.swapaxes(Wr, 1, 2)
    b1c = b1[:, :, None]
    b2c = b2[:, :, None]
    bfc = bf[:, :, None]
    bsc = bs[:, :, None]
    bvc = bv[:, :, None]
    brc = br[:, :, None]

    out_t = _grouped_mlp(sorted_tab.T, te, used,
                         W1t, b1c, W2t, b2c, Wft, bfc, Wst, bsc,
                         Wvt, bvc, Wrt, brc)

    out_rows = out_t.T
    final = out_rows[pos]

    rgb = final[:, 0:3].reshape(N_rays, N_samp, 3)
    sigma = final[:, 3:4].reshape(N_rays, N_samp, 1)
    return rgb, sigma


# speedup vs baseline: 6.8900x; 1.3056x over previous
import functools

import jax
import jax.numpy as jnp
from jax import lax
from jax.experimental import pallas as pl
import jax.experimental.pallas.tpu as pltpu
from jax.experimental.pallas import tpu_sc as plsc

RES = 4
L_PTS = 10
L_DIR = 4
HID = 32
E = 64
TSZ = 256
TSZ_LOG = 8

NC, NS, LANES = 2, 16, 16
NW = NC * NS


def _worker_id():
    return lax.axis_index("s") * NC + lax.axis_index("c")


def _vox_from_xyz(xx, yy, zz):
    def q(v):
        return jnp.minimum(jnp.maximum((v + 1.0) * (0.5 * RES), 0.0),
                           RES - 1.0).astype(jnp.int32)
    return q(xx) * (RES * RES) + q(yy) * RES + q(zz)


def _make_hist(N):
    CH = N // NW
    VPW = CH // LANES
    mesh = plsc.VectorSubcoreMesh(core_axis_name="c", subcore_axis_name="s",
                                  num_cores=NC, num_subcores=NS)

    @functools.partial(
        pl.kernel, mesh=mesh,
        compiler_params=pltpu.CompilerParams(use_tc_tiling_on_sc=False, needs_layout_passes=False),
        out_type=jax.ShapeDtypeStruct((NW * E,), jnp.int32),
        scratch_types=[
            pltpu.VMEM((CH,), jnp.float32),
            pltpu.VMEM((CH,), jnp.float32),
            pltpu.VMEM((CH,), jnp.float32),
            pltpu.VMEM((CH,), jnp.int32),
            pltpu.VMEM((E,), jnp.int32),
        ],
    )
    def hist_kernel(x_hbm, y_hbm, z_hbm, hist_hbm, xv, yv, zv, voxv, histv):
        wid = _worker_id()
        base = wid * CH
        pltpu.sync_copy(x_hbm.at[pl.ds(base, CH)], xv)
        pltpu.sync_copy(y_hbm.at[pl.ds(base, CH)], yv)
        pltpu.sync_copy(z_hbm.at[pl.ds(base, CH)], zv)

        def vox_body(j, _):
            sl = pl.ds(j * LANES, LANES)
            voxv[sl] = _vox_from_xyz(xv[sl], yv[sl], zv[sl])
            return 0
        lax.fori_loop(0, VPW, vox_body, 0)

        def b_body(b, bvec):
            def j_body(j, cnt):
                m = voxv[pl.ds(j * LANES, LANES)] == bvec
                return cnt + jnp.sum(m.astype(jnp.int32))
            cnt = lax.fori_loop(0, VPW, j_body, jnp.int32(0))
            plsc.store_scatter(histv, [bvec],
                               jnp.broadcast_to(cnt, (LANES,)))
            return bvec + 1
        lax.fori_loop(0, E, b_body, jnp.zeros((LANES,), jnp.int32))
        pltpu.sync_copy(histv, hist_hbm.at[pl.ds(base // CH * E, E)])

    return hist_kernel


def _make_route(N, n_pad, tiles):
    CH = N // NW
    VPW = CH // LANES
    KCH = CH // 128
    TV = tiles // LANES
    mesh = plsc.VectorSubcoreMesh(core_axis_name="c", subcore_axis_name="s",
                                  num_cores=NC, num_subcores=NS)

    @functools.partial(
        pl.kernel, mesh=mesh,
        compiler_params=pltpu.CompilerParams(use_tc_tiling_on_sc=False, needs_layout_passes=False),
        out_type=[
            jax.ShapeDtypeStruct((n_pad, 8), jnp.float32),
            jax.ShapeDtypeStruct((N,), jnp.int32),
            jax.ShapeDtypeStruct((tiles,), jnp.int32),
            jax.ShapeDtypeStruct((LANES,), jnp.int32),
        ],
        scratch_types=[
            pltpu.VMEM((CH,), jnp.float32),
            pltpu.VMEM((CH,), jnp.float32),
            pltpu.VMEM((CH,), jnp.float32),
            pltpu.VMEM((CH,), jnp.int32),
            pltpu.VMEM((CH, 8), jnp.float32),
            pltpu.VMEM((NW * E,), jnp.int32),
            pltpu.VMEM((E,), jnp.int32),
            pltpu.VMEM((E,), jnp.int32),
            pltpu.VMEM((E,), jnp.int32),
            pltpu.VMEM((E,), jnp.int32),
            pltpu.VMEM((E,), jnp.int32),
            pltpu.VMEM((KCH, 128), jnp.int32),
            pltpu.VMEM((tiles,), jnp.int32),
            pltpu.VMEM((LANES,), jnp.int32),
            pltpu.SemaphoreType.DMA,
        ],
    )
    def route_kernel(x_hbm, y_hbm, z_hbm, comb_hbm, hist_hbm,
                     sorted_hbm, pos_hbm, te_hbm, used_hbm,
                     xv, yv, zv, voxv, combv, histv,
                     totv, segv, tstartv, pstartv, priorv,
                     pos3, tev, usedv, sem):
        wid = _worker_id()
        base = wid * CH
        pltpu.sync_copy(x_hbm.at[pl.ds(base, CH)], xv)
        pltpu.sync_copy(y_hbm.at[pl.ds(base, CH)], yv)
        pltpu.sync_copy(z_hbm.at[pl.ds(base, CH)], zv)
        pltpu.sync_copy(comb_hbm.at[pl.ds(base, CH)], combv)
        pltpu.sync_copy(hist_hbm, histv)

        def vox_body(j, _):
            sl = pl.ds(j * LANES, LANES)
            voxv[sl] = _vox_from_xyz(xv[sl], yv[sl], zv[sl])
            return 0
        lax.fori_loop(0, VPW, vox_body, 0)

        for k in range(E // LANES):
            sl = pl.ds(k * LANES, LANES)

            def tot_body(w, acc):
                return acc + histv[pl.ds(w * E + k * LANES, LANES)]
            totv[sl] = lax.fori_loop(0, NW, tot_body,
                                     jnp.zeros((LANES,), jnp.int32))
            priorv[sl] = lax.fori_loop(0, wid, tot_body,
                                       jnp.zeros((LANES,), jnp.int32))

        carry = jnp.int32(0)
        for k in range(E // LANES):
            sl = pl.ds(k * LANES, LANES)
            seg = (totv[sl] + (TSZ - 1)) >> TSZ_LOG
            segv[sl] = seg
            incl = plsc.cumsum(seg)
            tstartv[sl] = incl - seg + carry
            pstartv[sl] = (incl - seg + carry) * TSZ
            carry = carry + jnp.sum(seg)

        def b_body(b, bvec):
            bucket_base = (plsc.load_gather(pstartv, [bvec])
                           + plsc.load_gather(priorv, [bvec]))

            def j_body(j, run):
                r = j // (128 // LANES)
                sl = pl.ds((j % (128 // LANES)) * LANES, LANES)
                m = voxv[pl.ds(j * LANES, LANES)] == bvec
                mi = m.astype(jnp.int32)
                excl = plsc.cumsum(mi) - mi
                pos3[r, sl] = jnp.where(m, bucket_base + (excl + run),
                                        pos3[r, sl])
                return run + jnp.sum(mi)
            lax.fori_loop(0, VPW, j_body, jnp.int32(0))
            return bvec + 1
        lax.fori_loop(0, E, b_body, jnp.zeros((LANES,), jnp.int32))

        for k in range(KCH):
            pltpu.sync_copy(pos3.at[k], pos_hbm.at[pl.ds(base + k * 128, 128)])
        for k in range(KCH):
            pltpu.async_copy(combv.at[pl.ds(k * 128, 128)],
                             sorted_hbm.at[pos3.at[k]], sem).wait()

        @pl.when(wid == 0)
        def _te():
            def t_body(tk, tbase):
                tvec = lax.iota(jnp.int32, LANES) + tbase

                def b2_body(b, carry2):
                    cnt, bvec = carry2
                    g = plsc.load_gather(tstartv, [bvec])
                    return (cnt + (g <= tvec).astype(jnp.int32), bvec + 1)
                cnt, _ = lax.fori_loop(
                    0, E, b2_body,
                    (jnp.zeros((LANES,), jnp.int32),
                     jnp.zeros((LANES,), jnp.int32)))
                tev[pl.ds(tk * LANES, LANES)] = cnt - 1
                return tbase + LANES
            lax.fori_loop(0, TV, t_body, jnp.zeros((LANES,), jnp.int32))
            last = jnp.full((LANES,), E - 1, jnp.int32)
            usedv[...] = (plsc.load_gather(tstartv, [last])
                          + plsc.load_gather(segv, [last]))
            pltpu.sync_copy(tev, te_hbm)
            pltpu.sync_copy(usedv, used_hbm)

    return route_kernel


def _make_unsort(N, n_pad):
    CH = N // NW
    KCH = CH // 128
    mesh = plsc.VectorSubcoreMesh(core_axis_name="c", subcore_axis_name="s",
                                  num_cores=NC, num_subcores=NS)

    @functools.partial(
        pl.kernel, mesh=mesh,
        compiler_params=pltpu.CompilerParams(use_tc_tiling_on_sc=False, needs_layout_passes=False),
        out_type=jax.ShapeDtypeStruct((N, 8), jnp.float32),
        scratch_types=[
            pltpu.VMEM((KCH, 128), jnp.int32),
            pltpu.VMEM((CH, 8), jnp.float32),
            pltpu.SemaphoreType.DMA,
        ],
    )
    def unsort_kernel(rows_hbm, pos_hbm, final_hbm, pos3, rowsv, sem):
        wid = _worker_id()
        base = wid * CH
        for k in range(KCH):
            pltpu.sync_copy(pos_hbm.at[pl.ds(base + k * 128, 128)],
                            pos3.at[k])
        for k in range(KCH):
            pltpu.async_copy(rows_hbm.at[pos3.at[k]],
                             rowsv.at[pl.ds(k * 128, 128)], sem).wait()
        pltpu.sync_copy(rowsv, final_hbm.at[pl.ds(base, CH)])

    return unsort_kernel


def _pe_parts_t(x, L):
    parts = [x]
    for l in range(L):
        xb = x * (2.0 ** l)
        parts.append(jnp.sin(xb))
        parts.append(jnp.cos(xb))
    return jnp.concatenate(parts, axis=0)


def _mlp_kernel(te_ref, used_ref, rows_ref,
                W1_ref, b1_ref, W2_ref, b2_ref, Wf_ref, bf_ref,
                Ws_ref, bs_ref, Wv_ref, bv_ref, Wr_ref, br_ref,
                out_ref):
    t = pl.program_id(0)

    @pl.when(t < used_ref[0])
    def _compute():
        rt = rows_ref[...]
        x = rt[0:3, :]
        d = rt[3:6, :]
        emb_p = _pe_parts_t(x, L_PTS)
        emb_d = _pe_parts_t(d, L_DIR)

        dot = lambda a, b: jnp.dot(a, b, preferred_element_type=jnp.float32)
        h = jax.nn.relu(dot(W1_ref[0], emb_p) + b1_ref[0])
        h = jax.nn.relu(dot(W2_ref[0], h) + b2_ref[0])
        sig = dot(Ws_ref[0], h) + bs_ref[0]
        feat = dot(Wf_ref[0], h) + bf_ref[0]
        h2 = jnp.concatenate([feat, emb_d], axis=0)
        h2 = jax.nn.relu(dot(Wv_ref[0], h2) + bv_ref[0])
        rgb = dot(Wr_ref[0], h2) + br_ref[0]

        out_ref[...] = jnp.concatenate(
            [rgb, sig, jnp.zeros((4, rt.shape[1]), jnp.float32)], axis=0)


def _grouped_mlp(sorted_t, te, used,
                 W1t, b1c, W2t, b2c, Wft, bfc, Wst, bsc, Wvt, bvc, Wrt, brc):
    n_pad = sorted_t.shape[1]
    tiles = n_pad // TSZ

    D_P = 3 + 6 * L_PTS
    D_D = 3 + 6 * L_DIR

    blk = pl.BlockSpec((8, TSZ), lambda t, te_r, used_r: (0, t))
    per_e = lambda d0, d1: pl.BlockSpec(
        (1, d0, d1), lambda t, te_r, used_r: (te_r[t], 0, 0))

    grid_spec = pltpu.PrefetchScalarGridSpec(
        num_scalar_prefetch=2,
        grid=(tiles,),
        in_specs=[
            blk,
            per_e(HID, D_P), per_e(HID, 1),
            per_e(HID, HID), per_e(HID, 1),
            per_e(HID, HID), per_e(HID, 1),
            per_e(1, HID), per_e(1, 1),
            per_e(HID, D_D + HID), per_e(HID, 1),
            per_e(3, HID), per_e(3, 1),
        ],
        out_specs=blk,
    )
    return pl.pallas_call(
        _mlp_kernel,
        grid_spec=grid_spec,
        out_shape=jax.ShapeDtypeStruct((8, n_pad), jnp.float32),
    )(te, used, sorted_t,
      W1t, b1c, W2t, b2c, Wft, bfc, Wst, bsc, Wvt, bvc, Wrt, brc)


def kernel(pts, viewdirs, W1, b1, W2, b2, Wf, bf, Ws, bs, Wv, bv, Wr, br):
    N_rays, N_samp, _ = pts.shape
    N = N_rays * N_samp
    pts_flat = pts.reshape(N, 3)
    dirs_flat = jnp.broadcast_to(viewdirs[:, None, :], (N_rays, N_samp, 3)).reshape(N, 3)

    tiles = N // TSZ + E
    n_pad = tiles * TSZ

    x = pts_flat[:, 0]
    y = pts_flat[:, 1]
    z = pts_flat[:, 2]
    comb = jnp.concatenate(
        [pts_flat, dirs_flat, jnp.zeros((N, 2), jnp.float32)], axis=1)

    hist = _make_hist(N)(x, y, z)
    sorted_tab, pos, te, used = _make_route(N, n_pad, tiles)(
        x, y, z, comb, hist)

    W1t = jnp.swapaxes(W1, 1, 2)
    W2t = jnp.swapaxes(W2, 1, 2)
    Wft = jnp.swapaxes(Wf, 1, 2)
    Wst = jnp.swapaxes(Ws, 1, 2)
    Wvt = jnp.swapaxes(Wv, 1, 2)
    Wrt = jnp.swapaxes(Wr, 1, 2)
    b1c = b1[:, :, None]
    b2c = b2[:, :, None]
    bfc = bf[:, :, None]
    bsc = bs[:, :, None]
    bvc = bv[:, :, None]
    brc = br[:, :, None]

    out_t = _grouped_mlp(sorted_tab.T, te, used,
                         W1t, b1c, W2t, b2c, Wft, bfc, Wst, bsc,
                         Wvt, bvc, Wrt, brc)

    final = _make_unsort(N, n_pad)(out_t.T, pos)

    rgb = final[:, 0:3].reshape(N_rays, N_samp, 3)
    sigma = final[:, 3:4].reshape(N_rays, N_samp, 1)
    return rgb, sigma
